# Initial kernel scaffold; baseline (speedup 1.0000x reference)
import jax
import jax.numpy as jnp
from jax.experimental import pallas as pl


def kernel(x, edge_index, W1, b1, W2, b2):
    raise NotImplementedError("write your pallas kernel here")


# speedup vs baseline: 75.2872x; 75.2872x over previous
import functools

import jax
import jax.numpy as jnp
from jax import lax
from jax.experimental import pallas as pl
from jax.experimental.pallas import tpu as pltpu
from jax.experimental.pallas import tpu_sc as plsc

N_TILES = 16
N_CORES = 2
LANE = 128


def _round_up(x, m):
    return (x + m - 1) // m * m


def _sc_mesh():
    return plsc.VectorSubcoreMesh(core_axis_name="c", subcore_axis_name="s")


def _make_deg_kernel(n_pad, r_pad, chunk):
    slice_len = n_pad // N_TILES
    rows_per_worker = r_pad // (N_TILES * N_CORES)
    n_chunks = rows_per_worker // chunk

    @functools.partial(
        pl.kernel,
        out_type=jax.ShapeDtypeStruct((N_CORES, n_pad), jnp.float32),
        mesh=_sc_mesh(),
        scratch_types=[
            pltpu.VMEM_SHARED((n_pad,), jnp.float32),
            pltpu.VMEM((chunk, LANE), jnp.int32),
            pltpu.VMEM((LANE,), jnp.float32),
        ],
    )
    def deg_kernel(dst_hbm, zeros_hbm, ones_hbm, out_hbm, deg_sh, dst_buf, ones_buf):
        c = lax.axis_index("c")
        s = lax.axis_index("s")
        wid = s * N_CORES + c
        tb = s * slice_len
        pltpu.sync_copy(zeros_hbm.at[pl.ds(tb, slice_len)],
                        deg_sh.at[pl.ds(tb, slice_len)])
        pltpu.sync_copy(ones_hbm, ones_buf)
        plsc.subcore_barrier()
        base = wid * rows_per_worker

        def body(i, carry):
            pltpu.sync_copy(dst_hbm.at[pl.ds(base + i * chunk, chunk)], dst_buf)
            for j in range(chunk):
                pltpu.sync_copy(ones_buf, deg_sh.at[dst_buf.at[j]], add=True)
            return carry

        lax.fori_loop(0, n_chunks, body, 0)
        plsc.subcore_barrier()
        pltpu.sync_copy(deg_sh.at[pl.ds(tb, slice_len)],
                        out_hbm.at[c].at[pl.ds(tb, slice_len)])

    return deg_kernel


def _make_agg_kernel(n_pad, r_pad, feat, chunk):
    slice_len = n_pad // N_TILES
    rows_per_worker = r_pad // (N_TILES * N_CORES)
    n_chunks = rows_per_worker // chunk

    @functools.partial(
        pl.kernel,
        out_type=jax.ShapeDtypeStruct((N_CORES, n_pad, feat), jnp.float32),
        mesh=_sc_mesh(),
        scratch_types=[
            pltpu.VMEM_SHARED((n_pad, feat), jnp.float32),
            pltpu.VMEM_SHARED((n_pad, feat), jnp.float32),
            pltpu.VMEM((chunk, LANE), jnp.int32),
            pltpu.VMEM((chunk, LANE), jnp.int32),
            pltpu.VMEM((LANE, feat), jnp.float32),
        ],
        compiler_params=pltpu.CompilerParams(use_tc_tiling_on_sc=False),
    )
    def agg_kernel(g_hbm, src_hbm, dst_hbm, zeros_hbm, out_hbm,
                   g_sh, acc_sh, src_buf, dst_buf, rows_buf):
        c = lax.axis_index("c")
        s = lax.axis_index("s")
        wid = s * N_CORES + c
        tb = s * slice_len
        pltpu.sync_copy(zeros_hbm.at[pl.ds(tb, slice_len)],
                        acc_sh.at[pl.ds(tb, slice_len)])
        pltpu.sync_copy(g_hbm.at[pl.ds(tb, slice_len)],
                        g_sh.at[pl.ds(tb, slice_len)])
        plsc.subcore_barrier()
        base = wid * rows_per_worker

        def body(i, carry):
            pltpu.sync_copy(src_hbm.at[pl.ds(base + i * chunk, chunk)], src_buf)
            pltpu.sync_copy(dst_hbm.at[pl.ds(base + i * chunk, chunk)], dst_buf)
            for j in range(chunk):
                pltpu.sync_copy(g_sh.at[src_buf.at[j]], rows_buf)
                pltpu.sync_copy(rows_buf, acc_sh.at[dst_buf.at[j]], add=True)
            return carry

        lax.fori_loop(0, n_chunks, body, 0)
        plsc.subcore_barrier()
        pltpu.sync_copy(acc_sh.at[pl.ds(tb, slice_len)],
                        out_hbm.at[c].at[pl.ds(tb, slice_len)])

    return agg_kernel


def _pad_cols(v, width):
    b, f = v.shape
    if f == width:
        return v
    return jnp.concatenate([v, jnp.zeros((b, width - f), v.dtype)], axis=1)


def _d1_body(degp_ref, x_ref, w1_ref, s_ref, g1_ref):
    deg = 1.0 + degp_ref[0] + degp_ref[1]
    s = lax.rsqrt(deg)
    s_ref[...] = s
    h = jnp.dot(x_ref[...], w1_ref[...], preferred_element_type=jnp.float32)
    g1_ref[...] = _pad_cols(s * h, g1_ref.shape[1])


def _d2_body(e1p_ref, g1_ref, s_ref, w2_ref, b1_ref, g2_ref):
    f1 = w2_ref.shape[0]
    s = s_ref[...]
    e1 = e1p_ref[0, :, :f1] + e1p_ref[1, :, :f1] + g1_ref[:, :f1]
    h1 = jnp.maximum(s * e1 + b1_ref[...], 0.0)
    g2 = s * jnp.dot(h1, w2_ref[...], preferred_element_type=jnp.float32)
    g2_ref[...] = _pad_cols(g2, g2_ref.shape[1])


def _d3_body(e2p_ref, g2_ref, s_ref, b2_ref, out_ref):
    f2 = out_ref.shape[1]
    logits = (s_ref[...] * (e2p_ref[0, :, :f2] + e2p_ref[1, :, :f2] + g2_ref[:, :f2])
              + b2_ref[...])
    m = jnp.max(logits, axis=1, keepdims=True)
    lse = m + jnp.log(jnp.sum(jnp.exp(logits - m), axis=1, keepdims=True))
    out_ref[...] = logits - lse


def _dense_call(body, grid, in_specs, out_specs, out_shape):
    return pl.pallas_call(body, grid=grid, in_specs=in_specs,
                          out_specs=out_specs, out_shape=out_shape)


def kernel(x, edge_index, W1, b1, W2, b2):
    n = x.shape[0]
    e = edge_index.shape[1]
    f1 = W1.shape[1]
    f2 = W2.shape[1]

    block = 6400
    n_pad = _round_up(n, block)
    grid_n = n_pad // block

    chunk = 8
    r = _round_up(e, LANE) // LANE
    r_pad = _round_up(r, N_TILES * N_CORES * chunk)
    pad_e = r_pad * LANE - e

    src = edge_index[0].astype(jnp.int32)
    dst = edge_index[1].astype(jnp.int32)
    pad_idx = jnp.arange(pad_e, dtype=jnp.int32) % LANE
    src2 = jnp.concatenate([src, jnp.zeros((pad_e,), jnp.int32)]).reshape(r_pad, LANE)
    dst2 = jnp.concatenate([dst, n + pad_idx]).reshape(r_pad, LANE)

    x_pad = jnp.pad(x, ((0, n_pad - n), (0, 0)))
    zeros1 = jnp.zeros((n_pad,), jnp.float32)
    ones_l = jnp.ones((LANE,), jnp.float32)

    deg_k = _make_deg_kernel(n_pad, r_pad, chunk)
    degp = deg_k(dst2, zeros1, ones_l)
    degp3 = degp.reshape(N_CORES, n_pad, 1)

    fp = 8
    s_arr, g1 = _dense_call(
        _d1_body, (grid_n,),
        [pl.BlockSpec((N_CORES, block, 1), lambda i: (0, i, 0)),
         pl.BlockSpec((block, x.shape[1]), lambda i: (i, 0)),
         pl.BlockSpec(W1.shape, lambda i: (0, 0))],
        [pl.BlockSpec((block, 1), lambda i: (i, 0)),
         pl.BlockSpec((block, fp), lambda i: (i, 0))],
        [jax.ShapeDtypeStruct((n_pad, 1), jnp.float32),
         jax.ShapeDtypeStruct((n_pad, fp), jnp.float32)],
    )(degp3, x_pad, W1)

    agg = _make_agg_kernel(n_pad, r_pad, fp, chunk)
    zeros_f = jnp.zeros((n_pad, fp), jnp.float32)
    e1p = agg(g1, src2, dst2, zeros_f)

    g2 = _dense_call(
        _d2_body, (grid_n,),
        [pl.BlockSpec((N_CORES, block, fp), lambda i: (0, i, 0)),
         pl.BlockSpec((block, fp), lambda i: (i, 0)),
         pl.BlockSpec((block, 1), lambda i: (i, 0)),
         pl.BlockSpec(W2.shape, lambda i: (0, 0)),
         pl.BlockSpec((1, f1), lambda i: (0, 0))],
        pl.BlockSpec((block, fp), lambda i: (i, 0)),
        jax.ShapeDtypeStruct((n_pad, fp), jnp.float32),
    )(e1p, g1, s_arr, W2, b1.reshape(1, f1))

    e2p = agg(g2, src2, dst2, zeros_f)

    out = _dense_call(
        _d3_body, (grid_n,),
        [pl.BlockSpec((N_CORES, block, fp), lambda i: (0, i, 0)),
         pl.BlockSpec((block, fp), lambda i: (i, 0)),
         pl.BlockSpec((block, 1), lambda i: (i, 0)),
         pl.BlockSpec((1, f2), lambda i: (0, 0))],
        pl.BlockSpec((block, f2), lambda i: (i, 0)),
        jax.ShapeDtypeStruct((n_pad, f2), jnp.float32),
    )(e2p, g2, s_arr, b2.reshape(1, f2))

    return out[:n]


# speedup vs baseline: 75.2938x; 1.0001x over previous
import functools

import jax
import jax.numpy as jnp
from jax import lax
from jax.experimental import pallas as pl
from jax.experimental.pallas import tpu as pltpu
from jax.experimental.pallas import tpu_sc as plsc

N_TILES = 16
N_CORES = 2
LANE = 128


def _round_up(x, m):
    return (x + m - 1) // m * m


def _sc_mesh():
    return plsc.VectorSubcoreMesh(core_axis_name="c", subcore_axis_name="s")


def _make_deg_kernel(n_pad, r_pad, chunk):
    slice_len = n_pad // N_TILES
    rows_per_worker = r_pad // (N_TILES * N_CORES)
    n_chunks = rows_per_worker // chunk

    @functools.partial(
        pl.kernel,
        out_type=jax.ShapeDtypeStruct((N_CORES, n_pad), jnp.float32),
        mesh=_sc_mesh(),
        scratch_types=[
            pltpu.VMEM_SHARED((n_pad,), jnp.float32),
            pltpu.VMEM((chunk, LANE), jnp.int32),
            pltpu.VMEM((LANE,), jnp.float32),
        ],
    )
    def deg_kernel(dst_hbm, zeros_hbm, ones_hbm, out_hbm, deg_sh, dst_buf, ones_buf):
        c = lax.axis_index("c")
        s = lax.axis_index("s")
        wid = s * N_CORES + c
        tb = s * slice_len
        pltpu.sync_copy(zeros_hbm.at[pl.ds(tb, slice_len)],
                        deg_sh.at[pl.ds(tb, slice_len)])
        pltpu.sync_copy(ones_hbm, ones_buf)
        plsc.subcore_barrier()
        base = wid * rows_per_worker

        def body(i, carry):
            pltpu.sync_copy(dst_hbm.at[pl.ds(base + i * chunk, chunk)], dst_buf)
            for j in range(chunk):
                pltpu.sync_copy(ones_buf, deg_sh.at[dst_buf.at[j]], add=True)
            return carry

        lax.fori_loop(0, n_chunks, body, 0)
        plsc.subcore_barrier()
        pltpu.sync_copy(deg_sh.at[pl.ds(tb, slice_len)],
                        out_hbm.at[c].at[pl.ds(tb, slice_len)])

    return deg_kernel


def _make_agg_kernel(n_pad, r_pad, feat, chunk):
    slice_len = n_pad // N_TILES
    rows_per_worker = r_pad // (N_TILES * N_CORES)
    n_chunks = rows_per_worker // chunk

    @functools.partial(
        pl.kernel,
        out_type=jax.ShapeDtypeStruct((N_CORES, n_pad, feat), jnp.float32),
        mesh=_sc_mesh(),
        scratch_types=[
            pltpu.VMEM_SHARED((n_pad, feat), jnp.float32),
            pltpu.VMEM_SHARED((n_pad, feat), jnp.float32),
            pltpu.VMEM((chunk, LANE), jnp.int32),
            pltpu.VMEM((chunk, LANE), jnp.int32),
            pltpu.VMEM((LANE, feat), jnp.float32),
        ],
        compiler_params=pltpu.CompilerParams(use_tc_tiling_on_sc=False),
    )
    def agg_kernel(g_hbm, src_hbm, dst_hbm, zeros_hbm, out_hbm,
                   g_sh, acc_sh, src_buf, dst_buf, rows_buf):
        c = lax.axis_index("c")
        s = lax.axis_index("s")
        wid = s * N_CORES + c
        tb = s * slice_len
        pltpu.sync_copy(zeros_hbm.at[pl.ds(tb, slice_len)],
                        acc_sh.at[pl.ds(tb, slice_len)])
        pltpu.sync_copy(g_hbm.at[pl.ds(tb, slice_len)],
                        g_sh.at[pl.ds(tb, slice_len)])
        plsc.subcore_barrier()
        base = wid * rows_per_worker

        def body(i, carry):
            pltpu.sync_copy(src_hbm.at[pl.ds(base + i * chunk, chunk)], src_buf)
            pltpu.sync_copy(dst_hbm.at[pl.ds(base + i * chunk, chunk)], dst_buf)
            for j in range(chunk):
                pltpu.sync_copy(g_sh.at[src_buf.at[j]], rows_buf)
                pltpu.sync_copy(rows_buf, acc_sh.at[dst_buf.at[j]], add=True)
            return carry

        lax.fori_loop(0, n_chunks, body, 0)
        plsc.subcore_barrier()
        pltpu.sync_copy(acc_sh.at[pl.ds(tb, slice_len)],
                        out_hbm.at[c].at[pl.ds(tb, slice_len)])

    return agg_kernel


def _pad_cols(v, width):
    b, f = v.shape
    if f == width:
        return v
    return jnp.concatenate([v, jnp.zeros((b, width - f), v.dtype)], axis=1)


def _d1_body(degp_ref, x_ref, w1_ref, s_ref, g1_ref):
    deg = 1.0 + degp_ref[0] + degp_ref[1]
    s = lax.rsqrt(deg)
    s_ref[...] = s
    h = jnp.dot(x_ref[...], w1_ref[...], preferred_element_type=jnp.float32)
    g1_ref[...] = _pad_cols(s * h, g1_ref.shape[1])


def _d2_body(e1p_ref, g1_ref, s_ref, w2_ref, b1_ref, g2_ref):
    f1 = w2_ref.shape[0]
    s = s_ref[...]
    e1 = e1p_ref[0, :, :f1] + e1p_ref[1, :, :f1] + g1_ref[:, :f1]
    h1 = jnp.maximum(s * e1 + b1_ref[...], 0.0)
    g2 = s * jnp.dot(h1, w2_ref[...], preferred_element_type=jnp.float32)
    g2_ref[...] = _pad_cols(g2, g2_ref.shape[1])


def _d3_body(e2p_ref, g2_ref, s_ref, b2_ref, out_ref):
    f2 = out_ref.shape[1]
    logits = (s_ref[...] * (e2p_ref[0, :, :f2] + e2p_ref[1, :, :f2] + g2_ref[:, :f2])
              + b2_ref[...])
    m = jnp.max(logits, axis=1, keepdims=True)
    lse = m + jnp.log(jnp.sum(jnp.exp(logits - m), axis=1, keepdims=True))
    out_ref[...] = logits - lse


def _dense_call(body, grid, in_specs, out_specs, out_shape):
    return pl.pallas_call(body, grid=grid, in_specs=in_specs,
                          out_specs=out_specs, out_shape=out_shape)


def kernel(x, edge_index, W1, b1, W2, b2):
    n = x.shape[0]
    e = edge_index.shape[1]
    f1 = W1.shape[1]
    f2 = W2.shape[1]

    block = 6400
    n_pad = _round_up(n, block)
    grid_n = n_pad // block

    chunk = 8
    r = _round_up(e, LANE) // LANE
    r_pad = _round_up(r, N_TILES * N_CORES * chunk)
    pad_e = r_pad * LANE - e

    src = edge_index[0].astype(jnp.int32)
    dst = edge_index[1].astype(jnp.int32)
    pad_idx = jnp.arange(pad_e, dtype=jnp.int32) % LANE
    src2 = jnp.concatenate([src, jnp.zeros((pad_e,), jnp.int32)]).reshape(r_pad, LANE)
    dst2 = jnp.concatenate([dst, n + pad_idx]).reshape(r_pad, LANE)

    x_pad = jnp.pad(x, ((0, n_pad - n), (0, 0)))
    zeros1 = jnp.zeros((n_pad,), jnp.float32)
    ones_l = jnp.ones((LANE,), jnp.float32)

    deg_k = _make_deg_kernel(n_pad, r_pad, chunk)
    degp = deg_k(dst2, zeros1, ones_l)
    degp3 = degp.reshape(N_CORES, n_pad, 1)

    fp1, fp2 = 8, 8
    s_arr, g1 = _dense_call(
        _d1_body, (grid_n,),
        [pl.BlockSpec((N_CORES, block, 1), lambda i: (0, i, 0)),
         pl.BlockSpec((block, x.shape[1]), lambda i: (i, 0)),
         pl.BlockSpec(W1.shape, lambda i: (0, 0))],
        [pl.BlockSpec((block, 1), lambda i: (i, 0)),
         pl.BlockSpec((block, fp1), lambda i: (i, 0))],
        [jax.ShapeDtypeStruct((n_pad, 1), jnp.float32),
         jax.ShapeDtypeStruct((n_pad, fp1), jnp.float32)],
    )(degp3, x_pad, W1)

    agg1 = _make_agg_kernel(n_pad, r_pad, fp1, chunk)
    e1p = agg1(g1, src2, dst2, jnp.zeros((n_pad, fp1), jnp.float32))

    g2 = _dense_call(
        _d2_body, (grid_n,),
        [pl.BlockSpec((N_CORES, block, fp1), lambda i: (0, i, 0)),
         pl.BlockSpec((block, fp1), lambda i: (i, 0)),
         pl.BlockSpec((block, 1), lambda i: (i, 0)),
         pl.BlockSpec(W2.shape, lambda i: (0, 0)),
         pl.BlockSpec((1, f1), lambda i: (0, 0))],
        pl.BlockSpec((block, fp2), lambda i: (i, 0)),
        jax.ShapeDtypeStruct((n_pad, fp2), jnp.float32),
    )(e1p, g1, s_arr, W2, b1.reshape(1, f1))

    agg2 = _make_agg_kernel(n_pad, r_pad, fp2, chunk)
    e2p = agg2(g2, src2, dst2, jnp.zeros((n_pad, fp2), jnp.float32))

    out = _dense_call(
        _d3_body, (grid_n,),
        [pl.BlockSpec((N_CORES, block, fp2), lambda i: (0, i, 0)),
         pl.BlockSpec((block, fp2), lambda i: (i, 0)),
         pl.BlockSpec((block, 1), lambda i: (i, 0)),
         pl.BlockSpec((1, f2), lambda i: (0, 0))],
        pl.BlockSpec((block, f2), lambda i: (i, 0)),
        jax.ShapeDtypeStruct((n_pad, f2), jnp.float32),
    )(e2p, g2, s_arr, b2.reshape(1, f2))

    return out[:n]


# speedup vs baseline: 96.8415x; 1.2862x over previous
import functools

import jax
import jax.numpy as jnp
from jax import lax
from jax.experimental import pallas as pl
from jax.experimental.pallas import tpu as pltpu
from jax.experimental.pallas import tpu_sc as plsc

N_TILES = 16
N_CORES = 2
LANE = 128


def _round_up(x, m):
    return (x + m - 1) // m * m


def _sc_mesh():
    return plsc.VectorSubcoreMesh(core_axis_name="c", subcore_axis_name="s")


def _make_deg_kernel(n_pad, r_pad, chunk):
    slice_len = n_pad // N_TILES
    rows_per_worker = r_pad // (N_TILES * N_CORES)
    n_chunks = rows_per_worker // chunk

    @functools.partial(
        pl.kernel,
        out_type=jax.ShapeDtypeStruct((N_CORES, n_pad), jnp.float32),
        mesh=_sc_mesh(),
        scratch_types=[
            pltpu.VMEM_SHARED((n_pad,), jnp.float32),
            pltpu.VMEM((chunk, LANE), jnp.int32),
            pltpu.VMEM((LANE,), jnp.float32),
            pltpu.SemaphoreType.DMA,
        ],
    )
    def deg_kernel(dst_hbm, zeros_hbm, ones_hbm, out_hbm, deg_sh, dst_buf, ones_buf,
                   ssem):
        c = lax.axis_index("c")
        s = lax.axis_index("s")
        wid = s * N_CORES + c
        tb = s * slice_len
        pltpu.sync_copy(zeros_hbm.at[pl.ds(tb, slice_len)],
                        deg_sh.at[pl.ds(tb, slice_len)])
        pltpu.sync_copy(ones_hbm, ones_buf)
        plsc.subcore_barrier()
        base = wid * rows_per_worker

        def body(i, carry):
            pltpu.sync_copy(dst_hbm.at[pl.ds(base + i * chunk, chunk)], dst_buf)
            descs = [pltpu.async_copy(ones_buf, deg_sh.at[dst_buf.at[j]], ssem,
                                      add=True)
                     for j in range(chunk)]
            for d in descs:
                d.wait()
            return carry

        lax.fori_loop(0, n_chunks, body, 0)
        plsc.subcore_barrier()
        pltpu.sync_copy(deg_sh.at[pl.ds(tb, slice_len)],
                        out_hbm.at[c].at[pl.ds(tb, slice_len)])

    return deg_kernel


def _make_agg_kernel(n_pad, r_pad, feat, chunk):
    slice_len = n_pad // N_TILES
    rows_per_worker = r_pad // (N_TILES * N_CORES)
    n_chunks = rows_per_worker // chunk

    @functools.partial(
        pl.kernel,
        out_type=jax.ShapeDtypeStruct((N_CORES, n_pad, feat), jnp.float32),
        mesh=_sc_mesh(),
        scratch_types=[
            pltpu.VMEM_SHARED((n_pad, feat), jnp.float32),
            pltpu.VMEM_SHARED((n_pad, feat), jnp.float32),
            pltpu.VMEM((chunk, LANE), jnp.int32),
            pltpu.VMEM((chunk, LANE), jnp.int32),
            pltpu.VMEM((chunk, LANE, feat), jnp.float32),
            pltpu.SemaphoreType.DMA,
            pltpu.SemaphoreType.DMA,
        ],
        compiler_params=pltpu.CompilerParams(use_tc_tiling_on_sc=False),
    )
    def agg_kernel(g_hbm, src_hbm, dst_hbm, zeros_hbm, out_hbm,
                   g_sh, acc_sh, src_buf, dst_buf, rows_buf, gsem, ssem):
        c = lax.axis_index("c")
        s = lax.axis_index("s")
        wid = s * N_CORES + c
        tb = s * slice_len
        pltpu.sync_copy(zeros_hbm.at[pl.ds(tb, slice_len)],
                        acc_sh.at[pl.ds(tb, slice_len)])
        pltpu.sync_copy(g_hbm.at[pl.ds(tb, slice_len)],
                        g_sh.at[pl.ds(tb, slice_len)])
        plsc.subcore_barrier()
        base = wid * rows_per_worker

        def body(i, carry):
            pltpu.sync_copy(src_hbm.at[pl.ds(base + i * chunk, chunk)], src_buf)
            pltpu.sync_copy(dst_hbm.at[pl.ds(base + i * chunk, chunk)], dst_buf)
            gd = [pltpu.async_copy(g_sh.at[src_buf.at[j]], rows_buf.at[j], gsem)
                  for j in range(chunk)]
            for d in gd:
                d.wait()
            sd = [pltpu.async_copy(rows_buf.at[j], acc_sh.at[dst_buf.at[j]], ssem,
                                   add=True)
                  for j in range(chunk)]
            for d in sd:
                d.wait()
            return carry

        lax.fori_loop(0, n_chunks, body, 0)
        plsc.subcore_barrier()
        pltpu.sync_copy(acc_sh.at[pl.ds(tb, slice_len)],
                        out_hbm.at[c].at[pl.ds(tb, slice_len)])

    return agg_kernel


def _pad_cols(v, width):
    b, f = v.shape
    if f == width:
        return v
    return jnp.concatenate([v, jnp.zeros((b, width - f), v.dtype)], axis=1)


def _d1_body(degp_ref, x_ref, w1_ref, s_ref, g1_ref):
    deg = 1.0 + degp_ref[0] + degp_ref[1]
    s = lax.rsqrt(deg)
    s_ref[...] = s
    h = jnp.dot(x_ref[...], w1_ref[...], preferred_element_type=jnp.float32)
    g1_ref[...] = _pad_cols(s * h, g1_ref.shape[1])


def _d2_body(e1p_ref, g1_ref, s_ref, w2_ref, b1_ref, g2_ref):
    f1 = w2_ref.shape[0]
    s = s_ref[...]
    e1 = e1p_ref[0, :, :f1] + e1p_ref[1, :, :f1] + g1_ref[:, :f1]
    h1 = jnp.maximum(s * e1 + b1_ref[...], 0.0)
    g2 = s * jnp.dot(h1, w2_ref[...], preferred_element_type=jnp.float32)
    g2_ref[...] = _pad_cols(g2, g2_ref.shape[1])


def _d3_body(e2p_ref, g2_ref, s_ref, b2_ref, out_ref):
    f2 = out_ref.shape[1]
    logits = (s_ref[...] * (e2p_ref[0, :, :f2] + e2p_ref[1, :, :f2] + g2_ref[:, :f2])
              + b2_ref[...])
    m = jnp.max(logits, axis=1, keepdims=True)
    lse = m + jnp.log(jnp.sum(jnp.exp(logits - m), axis=1, keepdims=True))
    out_ref[...] = logits - lse


def _dense_call(body, grid, in_specs, out_specs, out_shape):
    return pl.pallas_call(body, grid=grid, in_specs=in_specs,
                          out_specs=out_specs, out_shape=out_shape)


def kernel(x, edge_index, W1, b1, W2, b2):
    n = x.shape[0]
    e = edge_index.shape[1]
    f1 = W1.shape[1]
    f2 = W2.shape[1]

    block = 6400
    n_pad = _round_up(n, block)
    grid_n = n_pad // block

    chunk = 8
    r = _round_up(e, LANE) // LANE
    r_pad = _round_up(r, N_TILES * N_CORES * chunk)
    pad_e = r_pad * LANE - e

    src = edge_index[0].astype(jnp.int32)
    dst = edge_index[1].astype(jnp.int32)
    pad_idx = jnp.arange(pad_e, dtype=jnp.int32) % LANE
    src2 = jnp.concatenate([src, jnp.zeros((pad_e,), jnp.int32)]).reshape(r_pad, LANE)
    dst2 = jnp.concatenate([dst, n + pad_idx]).reshape(r_pad, LANE)

    x_pad = jnp.pad(x, ((0, n_pad - n), (0, 0)))
    zeros1 = jnp.zeros((n_pad,), jnp.float32)
    ones_l = jnp.ones((LANE,), jnp.float32)

    deg_k = _make_deg_kernel(n_pad, r_pad, chunk)
    degp = deg_k(dst2, zeros1, ones_l)
    degp3 = degp.reshape(N_CORES, n_pad, 1)

    fp1, fp2 = 8, 8
    s_arr, g1 = _dense_call(
        _d1_body, (grid_n,),
        [pl.BlockSpec((N_CORES, block, 1), lambda i: (0, i, 0)),
         pl.BlockSpec((block, x.shape[1]), lambda i: (i, 0)),
         pl.BlockSpec(W1.shape, lambda i: (0, 0))],
        [pl.BlockSpec((block, 1), lambda i: (i, 0)),
         pl.BlockSpec((block, fp1), lambda i: (i, 0))],
        [jax.ShapeDtypeStruct((n_pad, 1), jnp.float32),
         jax.ShapeDtypeStruct((n_pad, fp1), jnp.float32)],
    )(degp3, x_pad, W1)

    agg1 = _make_agg_kernel(n_pad, r_pad, fp1, chunk)
    e1p = agg1(g1, src2, dst2, jnp.zeros((n_pad, fp1), jnp.float32))

    g2 = _dense_call(
        _d2_body, (grid_n,),
        [pl.BlockSpec((N_CORES, block, fp1), lambda i: (0, i, 0)),
         pl.BlockSpec((block, fp1), lambda i: (i, 0)),
         pl.BlockSpec((block, 1), lambda i: (i, 0)),
         pl.BlockSpec(W2.shape, lambda i: (0, 0)),
         pl.BlockSpec((1, f1), lambda i: (0, 0))],
        pl.BlockSpec((block, fp2), lambda i: (i, 0)),
        jax.ShapeDtypeStruct((n_pad, fp2), jnp.float32),
    )(e1p, g1, s_arr, W2, b1.reshape(1, f1))

    agg2 = _make_agg_kernel(n_pad, r_pad, fp2, chunk)
    e2p = agg2(g2, src2, dst2, jnp.zeros((n_pad, fp2), jnp.float32))

    out = _dense_call(
        _d3_body, (grid_n,),
        [pl.BlockSpec((N_CORES, block, fp2), lambda i: (0, i, 0)),
         pl.BlockSpec((block, fp2), lambda i: (i, 0)),
         pl.BlockSpec((block, 1), lambda i: (i, 0)),
         pl.BlockSpec((1, f2), lambda i: (0, 0))],
        pl.BlockSpec((block, f2), lambda i: (i, 0)),
        jax.ShapeDtypeStruct((n_pad, f2), jnp.float32),
    )(e2p, g2, s_arr, b2.reshape(1, f2))

    return out[:n]
